# baseline (speedup 1.0000x reference)
import functools

import jax
import jax.numpy as jnp
import numpy as np
from jax.experimental import pallas as pl
from jax.experimental.pallas import tpu as pltpu

_EPS = 1e-5
_R1SQ = 0.2 * 0.2
_R2SQ = 0.4 * 0.4
_K = 64
_M1 = 512
_M2 = 128
_CTILE = 128
_PTILE = 1024
_F32 = jnp.float32


def _mm(w, x):
    return jax.lax.dot_general(w, x, (((1,), (0,)), ((), ())),
                               preferred_element_type=_F32)


def _cumsum_flat(x):
    r, n = x.shape
    s = 1
    while s < n:
        x = x + jnp.concatenate(
            [jnp.zeros((r, s), _F32), x[:, :n - s]], axis=-1)
        s *= 2
    return x


def _cumsum_lanes(x):
    r, n = x.shape
    if n <= 128:
        return _cumsum_flat(x)
    nc = n // 128
    xr = x.reshape(r, nc, 128)
    s = 1
    while s < 128:
        xr = xr + jnp.concatenate(
            [jnp.zeros((r, nc, s), _F32), xr[:, :, :128 - s]], axis=-1)
        s *= 2
    tot = xr[:, :, 127:128]
    off = jnp.concatenate(
        [jnp.zeros((r, 1, 1), _F32), tot[:, :nc - 1, :]], axis=1)
    s = 1
    while s < nc:
        off = off + jnp.concatenate(
            [jnp.zeros((r, s, 1), _F32), off[:, :nc - s, :]], axis=1)
        s *= 2
    return (xr + off).reshape(r, n)


def _mm_t(a, b):
    return jax.lax.dot_general(a, b, (((1,), (1,)), ((), ())),
                               preferred_element_type=_F32)


def _fps_kernel(xr_ref, nx_ref, *, m):
    ncol = xr_ref.shape[2]
    x0 = xr_ref[0, 0:8, :]
    x1 = xr_ref[0, 8:16, :]
    x2 = xr_ref[0, 16:24, :]
    lin = (jax.lax.broadcasted_iota(jnp.int32, (8, ncol), 0) * ncol
           + jax.lax.broadcasted_iota(jnp.int32, (8, ncol), 1))
    lane_m = jax.lax.broadcasted_iota(jnp.int32, (1, m), 1)

    def body(i, carry):
        dists, far, nx = carry
        oh = (lin == far).astype(_F32)
        c0 = jnp.sum(x0 * oh).reshape(1, 1)
        c1 = jnp.sum(x1 * oh).reshape(1, 1)
        c2 = jnp.sum(x2 * oh).reshape(1, 1)
        ccol = jnp.concatenate([c0, c1, c2], axis=0)
        nx = nx + ccol * (lane_m == i).astype(_F32)
        d = (x0 - c0) ** 2 + (x1 - c1) ** 2 + (x2 - c2) ** 2
        dists = jnp.minimum(dists, d)
        gmax = jnp.max(dists)
        far = jnp.min(jnp.where(dists == gmax, lin, jnp.int32(2 ** 30)))
        return dists, far, nx

    d0 = jnp.full((8, ncol), 1e10, _F32)
    nx0 = jnp.zeros((3, m), _F32)
    _, _, nx = jax.lax.fori_loop(0, m, body, (d0, jnp.int32(0), nx0))
    nx_ref[0] = nx


def _fps_call(xyz, m):
    b, _, n = xyz.shape
    xr = jnp.reshape(xyz, (b, 24, n // 8))
    return pl.pallas_call(
        functools.partial(_fps_kernel, m=m),
        grid=(b,),
        in_specs=[pl.BlockSpec((1, 24, n // 8), lambda i: (i, 0, 0))],
        out_specs=pl.BlockSpec((1, 3, m), lambda i: (i, 0, 0)),
        out_shape=jax.ShapeDtypeStruct((b, 3, m), _F32),
        compiler_params=pltpu.CompilerParams(
            dimension_semantics=("parallel",)),
    )(xr)


def _sa_kernel(xyz_ref, nx_ref, nxT_ref, feat_ref, *wref, r2, k, cin, ct,
               has_feat):
    ws = wref[:-2]
    out_ref, scr = wref[-2], wref[-1]
    xyz = xyz_ref[0]
    n = xyz.shape[1]
    nxT = nxT_ref[0]
    px0 = xyz[0:1, :]
    px1 = xyz[1:2, :]
    px2 = xyz[2:3, :]
    cx0 = nxT[:, 0:1]
    cx1 = nxT[:, 1:2]
    cx2 = nxT[:, 2:3]
    d2 = (cx0 - px0) ** 2 + (cx1 - px1) ** 2 + (cx2 - px2) ** 2
    mask = (d2 < r2).astype(_F32)
    cnt = _cumsum_lanes(mask)
    sel = mask * cnt
    tot = cnt[:, n - 1:n]
    tot_row = jax.lax.transpose(tot, (1, 0))
    nx = nx_ref[0]
    if has_feat:
        feat = feat_ref[0]

    tile_max = jnp.max(tot)

    oh0 = (sel == 1.0).astype(_F32)
    g0 = _mm_t(xyz, oh0)
    gfill = g0 - nx
    scr[0:3, 0:ct] = gfill
    if has_feat:
        f0 = _mm_t(feat, oh0)
        scr[3:, 0:ct] = f0
    else:
        f0 = None

    def slot(s, _):
        sf = s.astype(_F32)

        @pl.when(sf < tile_max)
        def _live():
            oh = (sel == sf + 1.0).astype(_F32)
            gt = _mm_t(xyz, oh)
            valid = tot_row > sf
            scr[0:3, pl.ds(s * ct, ct)] = jnp.where(valid, gt - nx, gfill)
            if has_feat:
                ft = _mm_t(feat, oh)
                scr[3:, pl.ds(s * ct, ct)] = jnp.where(valid, ft, f0)

        @pl.when(sf >= tile_max)
        def _fill():
            scr[0:3, pl.ds(s * ct, ct)] = gfill
            if has_feat:
                scr[3:, pl.ds(s * ct, ct)] = f0

        return 0

    jax.lax.fori_loop(1, k, slot, 0)

    x = scr[...]
    for wr, br in zip(ws[0::2], ws[1::2]):
        x = jnp.maximum(_mm(wr[...], x) + br[...], 0.0)
    cout = x.shape[0]
    out_ref[0] = jnp.max(x.reshape(cout, k, ct), axis=1)


def _sa_call(xyz, nx, nxT, feat, weights, r2, k, ct):
    b, _, n = xyz.shape
    m = nx.shape[2]
    has_feat = feat is not None
    cf = feat.shape[1] if has_feat else 0
    cin = 3 + cf
    cout = weights[-2].shape[0]
    nt = m // ct
    wspecs = [pl.BlockSpec(w.shape, lambda bb, t: (0, 0)) for w in weights]
    fspec = ([pl.BlockSpec((1, cf, n), lambda bb, t: (bb, 0, 0))]
             if has_feat else
             [pl.BlockSpec((1, 1, 1), lambda bb, t: (0, 0, 0))])
    farg = feat if has_feat else jnp.zeros((1, 1, 1), _F32)
    return pl.pallas_call(
        functools.partial(_sa_kernel, r2=r2, k=k, cin=cin, ct=ct,
                          has_feat=has_feat),
        grid=(b, nt),
        in_specs=[
            pl.BlockSpec((1, 3, n), lambda bb, t: (bb, 0, 0)),
            pl.BlockSpec((1, 3, ct), lambda bb, t: (bb, 0, t)),
            pl.BlockSpec((1, ct, 3), lambda bb, t: (bb, t, 0)),
        ] + fspec + wspecs,
        out_specs=pl.BlockSpec((1, cout, ct), lambda bb, t: (bb, 0, t)),
        out_shape=jax.ShapeDtypeStruct((b, cout, m), _F32),
        scratch_shapes=[pltpu.VMEM((cin, k * ct), _F32)],
        compiler_params=pltpu.CompilerParams(
            dimension_semantics=("parallel", "parallel")),
    )(xyz, nx, nxT, farg, *weights)


def _nn3_weights(d2):
    n, m = d2.shape
    wd = jnp.zeros((n, m), _F32)
    wsum = jnp.zeros((n, 1), _F32)
    lane = jax.lax.broadcasted_iota(jnp.int32, (n, m), 1)
    for _ in range(3):
        v = jnp.min(d2, axis=-1, keepdims=True)
        am = jnp.argmin(d2, axis=-1, keepdims=True)
        oh = (lane == am).astype(_F32)
        w = 1.0 / (v + 1e-8)
        wd = wd + w * oh
        wsum = wsum + w
        d2 = jnp.where(oh > 0.0, jnp.float32(np.inf), d2)
    return wd / wsum


def _d2_block(pcols, prows):
    return ((pcols[:, 0:1] - prows[0:1, :]) ** 2
            + (pcols[:, 1:2] - prows[1:2, :]) ** 2
            + (pcols[:, 2:3] - prows[2:3, :]) ** 2)


def _relu_mlp(x, ws):
    for wr, br in zip(ws[0::2], ws[1::2]):
        x = jnp.maximum(_mm(wr[...], x) + br[...], 0.0)
    return x


def _localfp1_kernel(nx2_ref, nx2T_ref, f2_ref, *refs, nl):
    lws = refs[:nl]
    pws = refs[nl:-1]
    out_ref = refs[-1]
    nx2 = nx2_ref[0]
    f2 = f2_ref[0]
    feat = jnp.concatenate([nx2, f2], axis=0)
    l3 = _relu_mlp(feat, lws)
    d2 = _d2_block(nx2T_ref[0], nx2)
    wd = _nn3_weights(d2)
    interp = _mm_t(l3, wd)
    new = jnp.concatenate([interp, f2], axis=0)
    out_ref[0] = _relu_mlp(new, pws)


def _localfp1_call(nx2, nx2T, f2, lweights, pweights):
    b = nx2.shape[0]
    m = nx2.shape[2]
    cout = pweights[-2].shape[0]
    ws = list(lweights) + list(pweights)
    wspecs = [pl.BlockSpec(w.shape, lambda bb: (0, 0)) for w in ws]
    return pl.pallas_call(
        functools.partial(_localfp1_kernel, nl=len(lweights)),
        grid=(b,),
        in_specs=[
            pl.BlockSpec((1, 3, m), lambda bb: (bb, 0, 0)),
            pl.BlockSpec((1, m, 3), lambda bb: (bb, 0, 0)),
            pl.BlockSpec((1,) + f2.shape[1:], lambda bb: (bb, 0, 0)),
        ] + wspecs,
        out_specs=pl.BlockSpec((1, cout, m), lambda bb: (bb, 0, 0)),
        out_shape=jax.ShapeDtypeStruct((b, cout, m), _F32),
        compiler_params=pltpu.CompilerParams(
            dimension_semantics=("parallel",)),
    )(nx2, nx2T, f2, *ws)


def _fp2_kernel(nx1_ref, nx1T_ref, nx2_ref, f1_ref, g1_ref, *refs):
    ws = refs[:-1]
    out_ref = refs[-1]
    d2 = _d2_block(nx1T_ref[0], nx2_ref[0])
    wd = _nn3_weights(d2)
    interp = _mm_t(g1_ref[0], wd)
    new = jnp.concatenate([interp, f1_ref[0]], axis=0)
    out_ref[0] = _relu_mlp(new, ws)


def _fp2_call(nx1, nx1T, nx2, f1, g1, weights):
    b, _, n = nx1.shape
    m = nx2.shape[2]
    cout = weights[-2].shape[0]
    wspecs = [pl.BlockSpec(w.shape, lambda bb: (0, 0)) for w in weights]
    return pl.pallas_call(
        _fp2_kernel,
        grid=(b,),
        in_specs=[
            pl.BlockSpec((1, 3, n), lambda bb: (bb, 0, 0)),
            pl.BlockSpec((1, n, 3), lambda bb: (bb, 0, 0)),
            pl.BlockSpec((1, 3, m), lambda bb: (bb, 0, 0)),
            pl.BlockSpec((1,) + f1.shape[1:], lambda bb: (bb, 0, 0)),
            pl.BlockSpec((1,) + g1.shape[1:], lambda bb: (bb, 0, 0)),
        ] + wspecs,
        out_specs=pl.BlockSpec((1, cout, n), lambda bb: (bb, 0, 0)),
        out_shape=jax.ShapeDtypeStruct((b, cout, n), _F32),
        compiler_params=pltpu.CompilerParams(
            dimension_semantics=("parallel",)),
    )(nx1, nx1T, nx2, f1, g1, *weights)


def _fp3seg_kernel(xyz_ref, xyzT_ref, nx1_ref, g2_ref, *refs, nf):
    fws = refs[:nf]
    sws = refs[nf:-3]
    lw_ref, lb_ref, out_ref = refs[-3], refs[-2], refs[-1]
    xyz = xyz_ref[0]
    d2 = _d2_block(xyzT_ref[0], nx1_ref[0])
    wd = _nn3_weights(d2)
    interp = _mm_t(g2_ref[0], wd)
    x = jnp.concatenate([interp, xyz], axis=0)
    x = _relu_mlp(x, fws)
    x = _relu_mlp(x, sws)
    out_ref[0] = _mm(lw_ref[...], x) + lb_ref[...]


def _fp3seg_call(xyz, xyzT, nx1, g2, fweights, sweights, lw, lb, pt):
    b, _, n = xyz.shape
    m = nx1.shape[2]
    nt = n // pt
    ws = list(fweights) + list(sweights) + [lw, lb]
    nseg = lw.shape[0]
    wspecs = [pl.BlockSpec(w.shape, lambda bb, t: (0, 0)) for w in ws]
    return pl.pallas_call(
        functools.partial(_fp3seg_kernel, nf=len(fweights)),
        grid=(b, nt),
        in_specs=[
            pl.BlockSpec((1, 3, pt), lambda bb, t: (bb, 0, t)),
            pl.BlockSpec((1, pt, 3), lambda bb, t: (bb, t, 0)),
            pl.BlockSpec((1, 3, m), lambda bb, t: (bb, 0, 0)),
            pl.BlockSpec((1,) + g2.shape[1:], lambda bb, t: (bb, 0, 0)),
        ] + wspecs,
        out_specs=pl.BlockSpec((1, nseg, pt), lambda bb, t: (bb, 0, t)),
        out_shape=jax.ShapeDtypeStruct((b, nseg, n), _F32),
        compiler_params=pltpu.CompilerParams(
            dimension_semantics=("parallel", "parallel")),
    )(xyz, xyzT, nx1, g2, *ws)


def _fold(layers):
    s = 1.0 / np.sqrt(1.0 + _EPS)
    out = []
    for w, g, bb in layers:
        out.append(w * (g * s)[:, None])
        out.append(bb[:, None])
    return out


def kernel(points, params):
    xyz0 = points[:, 0:3, :]
    xyz0T = jnp.transpose(xyz0, (0, 2, 1))

    sa1w = _fold(params['sa1'])
    sa2w = _fold(params['sa2'])
    locw = _fold(params['local'])
    fp1w = _fold(params['fp1'])
    fp2w = _fold(params['fp2'])
    fp3w = _fold(params['fp3'])
    segw = _fold(params['seg'])
    lw = params['logit_w']
    lb = params['logit_b'][:, None]

    nx1 = _fps_call(xyz0, _M1)
    nx1T = jnp.transpose(nx1, (0, 2, 1))
    f1 = _sa_call(xyz0, nx1, nx1T, None, sa1w, _R1SQ, _K, _CTILE)
    nx2 = _fps_call(nx1, _M2)
    nx2T = jnp.transpose(nx2, (0, 2, 1))
    f2 = _sa_call(nx1, nx2, nx2T, f1, sa2w, _R2SQ, _K, _M2)
    g1 = _localfp1_call(nx2, nx2T, f2, locw, fp1w)
    g2 = _fp2_call(nx1, nx1T, nx2, f1, g1, fp2w)
    return _fp3seg_call(xyz0, xyz0T, nx1, g2, fp3w, segw, lw, lb, _PTILE)


# speedup vs baseline: 1.0683x; 1.0683x over previous
import functools

import jax
import jax.numpy as jnp
import numpy as np
from jax.experimental import pallas as pl
from jax.experimental.pallas import tpu as pltpu

_EPS = 1e-5
_R1SQ = 0.2 * 0.2
_R2SQ = 0.4 * 0.4
_K = 64
_M1 = 512
_M2 = 128
_CTILE = 128
_PTILE = 2048
_F32 = jnp.float32


def _mm(w, x):
    return jax.lax.dot_general(w, x, (((1,), (0,)), ((), ())),
                               preferred_element_type=_F32)


def _cumsum_flat(x):
    r, n = x.shape
    s = 1
    while s < n:
        x = x + jnp.concatenate(
            [jnp.zeros((r, s), _F32), x[:, :n - s]], axis=-1)
        s *= 2
    return x


def _cumsum_lanes(x):
    return _cumsum_flat(x)


def _mm_t(a, b):
    return jax.lax.dot_general(a, b, (((1,), (1,)), ((), ())),
                               preferred_element_type=_F32)


def _fps_kernel(xr_ref, nx_ref, *, m):
    ncol = xr_ref.shape[2]
    x0 = xr_ref[0, 0:8, :]
    x1 = xr_ref[0, 8:16, :]
    x2 = xr_ref[0, 16:24, :]
    lin = (jax.lax.broadcasted_iota(jnp.int32, (8, ncol), 0) * ncol
           + jax.lax.broadcasted_iota(jnp.int32, (8, ncol), 1))
    lane_m = jax.lax.broadcasted_iota(jnp.int32, (1, m), 1)

    def body(i, carry):
        dists, far, nx = carry
        oh = (lin == far).astype(_F32)
        c0 = jnp.sum(x0 * oh).reshape(1, 1)
        c1 = jnp.sum(x1 * oh).reshape(1, 1)
        c2 = jnp.sum(x2 * oh).reshape(1, 1)
        ccol = jnp.concatenate([c0, c1, c2], axis=0)
        nx = nx + ccol * (lane_m == i).astype(_F32)
        d = (x0 - c0) ** 2 + (x1 - c1) ** 2 + (x2 - c2) ** 2
        dists = jnp.minimum(dists, d)
        gmax = jnp.max(dists)
        far = jnp.min(jnp.where(dists == gmax, lin, jnp.int32(2 ** 30)))
        return dists, far, nx

    d0 = jnp.full((8, ncol), 1e10, _F32)
    nx0 = jnp.zeros((3, m), _F32)
    _, _, nx = jax.lax.fori_loop(0, m, body, (d0, jnp.int32(0), nx0))
    nx_ref[0] = nx


def _fps_call(xyz, m):
    b, _, n = xyz.shape
    xr = jnp.reshape(xyz, (b, 24, n // 8))
    return pl.pallas_call(
        functools.partial(_fps_kernel, m=m),
        grid=(b,),
        in_specs=[pl.BlockSpec((1, 24, n // 8), lambda i: (i, 0, 0))],
        out_specs=pl.BlockSpec((1, 3, m), lambda i: (i, 0, 0)),
        out_shape=jax.ShapeDtypeStruct((b, 3, m), _F32),
        compiler_params=pltpu.CompilerParams(
            dimension_semantics=("parallel",)),
    )(xr)


def _sa_kernel(xyz_ref, nx_ref, nxT_ref, feat_ref, *wref, r2, k, cin, ct,
               has_feat):
    ws = wref[:-2]
    out_ref, scr = wref[-2], wref[-1]
    xyz = xyz_ref[0]
    n = xyz.shape[1]
    nxT = nxT_ref[0]
    px0 = xyz[0:1, :]
    px1 = xyz[1:2, :]
    px2 = xyz[2:3, :]
    cx0 = nxT[:, 0:1]
    cx1 = nxT[:, 1:2]
    cx2 = nxT[:, 2:3]
    d2 = (cx0 - px0) ** 2 + (cx1 - px1) ** 2 + (cx2 - px2) ** 2
    mask = (d2 < r2).astype(_F32)
    cnt = _cumsum_lanes(mask)
    sel = mask * cnt
    tot = cnt[:, n - 1:n]
    tot_row = jax.lax.transpose(tot, (1, 0))
    nx = nx_ref[0]
    if has_feat:
        feat = feat_ref[0]

    tile_max = jnp.max(tot)

    oh0 = (sel == 1.0).astype(_F32)
    g0 = _mm_t(xyz, oh0)
    gfill = g0 - nx
    scr[0:3, 0:ct] = gfill
    if has_feat:
        f0 = _mm_t(feat, oh0)
        scr[3:, 0:ct] = f0
    else:
        f0 = None

    def slot(s, _):
        sf = s.astype(_F32)

        @pl.when(sf < tile_max)
        def _live():
            oh = (sel == sf + 1.0).astype(_F32)
            gt = _mm_t(xyz, oh)
            valid = tot_row > sf
            scr[0:3, pl.ds(s * ct, ct)] = jnp.where(valid, gt - nx, gfill)
            if has_feat:
                ft = _mm_t(feat, oh)
                scr[3:, pl.ds(s * ct, ct)] = jnp.where(valid, ft, f0)

        @pl.when(sf >= tile_max)
        def _fill():
            scr[0:3, pl.ds(s * ct, ct)] = gfill
            if has_feat:
                scr[3:, pl.ds(s * ct, ct)] = f0

        return 0

    jax.lax.fori_loop(1, k, slot, 0)

    x = scr[...]
    for wr, br in zip(ws[0::2], ws[1::2]):
        x = jnp.maximum(_mm(wr[...], x) + br[...], 0.0)
    cout = x.shape[0]
    out_ref[0] = jnp.max(x.reshape(cout, k, ct), axis=1)


def _sa_call(xyz, nx, nxT, feat, weights, r2, k, ct):
    b, _, n = xyz.shape
    m = nx.shape[2]
    has_feat = feat is not None
    cf = feat.shape[1] if has_feat else 0
    cin = 3 + cf
    cout = weights[-2].shape[0]
    nt = m // ct
    wspecs = [pl.BlockSpec(w.shape, lambda bb, t: (0, 0)) for w in weights]
    fspec = ([pl.BlockSpec((1, cf, n), lambda bb, t: (bb, 0, 0))]
             if has_feat else
             [pl.BlockSpec((1, 1, 1), lambda bb, t: (0, 0, 0))])
    farg = feat if has_feat else jnp.zeros((1, 1, 1), _F32)
    return pl.pallas_call(
        functools.partial(_sa_kernel, r2=r2, k=k, cin=cin, ct=ct,
                          has_feat=has_feat),
        grid=(b, nt),
        in_specs=[
            pl.BlockSpec((1, 3, n), lambda bb, t: (bb, 0, 0)),
            pl.BlockSpec((1, 3, ct), lambda bb, t: (bb, 0, t)),
            pl.BlockSpec((1, ct, 3), lambda bb, t: (bb, t, 0)),
        ] + fspec + wspecs,
        out_specs=pl.BlockSpec((1, cout, ct), lambda bb, t: (bb, 0, t)),
        out_shape=jax.ShapeDtypeStruct((b, cout, m), _F32),
        scratch_shapes=[pltpu.VMEM((cin, k * ct), _F32)],
        compiler_params=pltpu.CompilerParams(
            dimension_semantics=("parallel", "parallel")),
    )(xyz, nx, nxT, farg, *weights)


def _nn3_weights(d2):
    n, m = d2.shape
    wd = jnp.zeros((n, m), _F32)
    wsum = jnp.zeros((n, 1), _F32)
    lane = jax.lax.broadcasted_iota(jnp.int32, (n, m), 1)
    for _ in range(3):
        v = jnp.min(d2, axis=-1, keepdims=True)
        am = jnp.argmin(d2, axis=-1, keepdims=True)
        oh = (lane == am).astype(_F32)
        w = 1.0 / (v + 1e-8)
        wd = wd + w * oh
        wsum = wsum + w
        d2 = jnp.where(oh > 0.0, jnp.float32(np.inf), d2)
    return wd / wsum


def _d2_block(pcols, prows):
    return ((pcols[:, 0:1] - prows[0:1, :]) ** 2
            + (pcols[:, 1:2] - prows[1:2, :]) ** 2
            + (pcols[:, 2:3] - prows[2:3, :]) ** 2)


def _relu_mlp(x, ws):
    for wr, br in zip(ws[0::2], ws[1::2]):
        x = jnp.maximum(_mm(wr[...], x) + br[...], 0.0)
    return x


def _localfp1_kernel(nx2_ref, nx2T_ref, f2_ref, *refs, nl):
    lws = refs[:nl]
    pws = refs[nl:-1]
    out_ref = refs[-1]
    nx2 = nx2_ref[0]
    f2 = f2_ref[0]
    feat = jnp.concatenate([nx2, f2], axis=0)
    l3 = _relu_mlp(feat, lws)
    d2 = _d2_block(nx2T_ref[0], nx2)
    wd = _nn3_weights(d2)
    interp = _mm_t(l3, wd)
    new = jnp.concatenate([interp, f2], axis=0)
    out_ref[0] = _relu_mlp(new, pws)


def _localfp1_call(nx2, nx2T, f2, lweights, pweights):
    b = nx2.shape[0]
    m = nx2.shape[2]
    cout = pweights[-2].shape[0]
    ws = list(lweights) + list(pweights)
    wspecs = [pl.BlockSpec(w.shape, lambda bb: (0, 0)) for w in ws]
    return pl.pallas_call(
        functools.partial(_localfp1_kernel, nl=len(lweights)),
        grid=(b,),
        in_specs=[
            pl.BlockSpec((1, 3, m), lambda bb: (bb, 0, 0)),
            pl.BlockSpec((1, m, 3), lambda bb: (bb, 0, 0)),
            pl.BlockSpec((1,) + f2.shape[1:], lambda bb: (bb, 0, 0)),
        ] + wspecs,
        out_specs=pl.BlockSpec((1, cout, m), lambda bb: (bb, 0, 0)),
        out_shape=jax.ShapeDtypeStruct((b, cout, m), _F32),
        compiler_params=pltpu.CompilerParams(
            dimension_semantics=("parallel",)),
    )(nx2, nx2T, f2, *ws)


def _fp2_kernel(nx1_ref, nx1T_ref, nx2_ref, f1_ref, g1_ref, *refs):
    ws = refs[:-1]
    out_ref = refs[-1]
    d2 = _d2_block(nx1T_ref[0], nx2_ref[0])
    wd = _nn3_weights(d2)
    interp = _mm_t(g1_ref[0], wd)
    new = jnp.concatenate([interp, f1_ref[0]], axis=0)
    out_ref[0] = _relu_mlp(new, ws)


def _fp2_call(nx1, nx1T, nx2, f1, g1, weights):
    b, _, n = nx1.shape
    m = nx2.shape[2]
    cout = weights[-2].shape[0]
    wspecs = [pl.BlockSpec(w.shape, lambda bb: (0, 0)) for w in weights]
    return pl.pallas_call(
        _fp2_kernel,
        grid=(b,),
        in_specs=[
            pl.BlockSpec((1, 3, n), lambda bb: (bb, 0, 0)),
            pl.BlockSpec((1, n, 3), lambda bb: (bb, 0, 0)),
            pl.BlockSpec((1, 3, m), lambda bb: (bb, 0, 0)),
            pl.BlockSpec((1,) + f1.shape[1:], lambda bb: (bb, 0, 0)),
            pl.BlockSpec((1,) + g1.shape[1:], lambda bb: (bb, 0, 0)),
        ] + wspecs,
        out_specs=pl.BlockSpec((1, cout, n), lambda bb: (bb, 0, 0)),
        out_shape=jax.ShapeDtypeStruct((b, cout, n), _F32),
        compiler_params=pltpu.CompilerParams(
            dimension_semantics=("parallel",)),
    )(nx1, nx1T, nx2, f1, g1, *weights)


def _fp3seg_kernel(xyz_ref, xyzT_ref, nx1_ref, g2_ref, *refs, nf):
    fws = refs[:nf]
    sws = refs[nf:-3]
    lw_ref, lb_ref, out_ref = refs[-3], refs[-2], refs[-1]
    xyz = xyz_ref[0]
    d2 = _d2_block(xyzT_ref[0], nx1_ref[0])
    wd = _nn3_weights(d2)
    interp = _mm_t(g2_ref[0], wd)
    x = jnp.concatenate([interp, xyz], axis=0)
    x = _relu_mlp(x, fws)
    x = _relu_mlp(x, sws)
    out_ref[0] = _mm(lw_ref[...], x) + lb_ref[...]


def _fp3seg_call(xyz, xyzT, nx1, g2, fweights, sweights, lw, lb, pt):
    b, _, n = xyz.shape
    m = nx1.shape[2]
    nt = n // pt
    ws = list(fweights) + list(sweights) + [lw, lb]
    nseg = lw.shape[0]
    wspecs = [pl.BlockSpec(w.shape, lambda bb, t: (0, 0)) for w in ws]
    return pl.pallas_call(
        functools.partial(_fp3seg_kernel, nf=len(fweights)),
        grid=(b, nt),
        in_specs=[
            pl.BlockSpec((1, 3, pt), lambda bb, t: (bb, 0, t)),
            pl.BlockSpec((1, pt, 3), lambda bb, t: (bb, t, 0)),
            pl.BlockSpec((1, 3, m), lambda bb, t: (bb, 0, 0)),
            pl.BlockSpec((1,) + g2.shape[1:], lambda bb, t: (bb, 0, 0)),
        ] + wspecs,
        out_specs=pl.BlockSpec((1, nseg, pt), lambda bb, t: (bb, 0, t)),
        out_shape=jax.ShapeDtypeStruct((b, nseg, n), _F32),
        compiler_params=pltpu.CompilerParams(
            dimension_semantics=("parallel", "parallel")),
    )(xyz, xyzT, nx1, g2, *ws)


def _fold(layers):
    s = 1.0 / np.sqrt(1.0 + _EPS)
    out = []
    for w, g, bb in layers:
        out.append(w * (g * s)[:, None])
        out.append(bb[:, None])
    return out


def kernel(points, params):
    xyz0 = points[:, 0:3, :]
    xyz0T = jnp.transpose(xyz0, (0, 2, 1))

    sa1w = _fold(params['sa1'])
    sa2w = _fold(params['sa2'])
    locw = _fold(params['local'])
    fp1w = _fold(params['fp1'])
    fp2w = _fold(params['fp2'])
    fp3w = _fold(params['fp3'])
    segw = _fold(params['seg'])
    lw = params['logit_w']
    lb = params['logit_b'][:, None]

    nx1 = _fps_call(xyz0, _M1)
    nx1T = jnp.transpose(nx1, (0, 2, 1))
    f1 = _sa_call(xyz0, nx1, nx1T, None, sa1w, _R1SQ, _K, _CTILE)
    nx2 = _fps_call(nx1, _M2)
    nx2T = jnp.transpose(nx2, (0, 2, 1))
    f2 = _sa_call(nx1, nx2, nx2T, f1, sa2w, _R2SQ, _K, _M2)
    g1 = _localfp1_call(nx2, nx2T, f2, locw, fp1w)
    g2 = _fp2_call(nx1, nx1T, nx2, f1, g1, fp2w)
    return _fp3seg_call(xyz0, xyz0T, nx1, g2, fp3w, segw, lw, lb, _PTILE)


# speedup vs baseline: 2.4217x; 2.2670x over previous
import functools

import jax
import jax.numpy as jnp
import numpy as np
from jax.experimental import pallas as pl
from jax.experimental.pallas import tpu as pltpu

_EPS = 1e-5
_R1SQ = 0.2 * 0.2
_R2SQ = 0.4 * 0.4
_K = 64
_M1 = 512
_M2 = 128
_CTILE = 128
_PTILE = 2048
_F32 = jnp.float32


def _mm(w, x):
    return jax.lax.dot_general(w, x, (((1,), (0,)), ((), ())),
                               preferred_element_type=_F32)


def _cumsum_flat(x):
    r, n = x.shape
    s = 1
    while s < n:
        x = x + jnp.concatenate(
            [jnp.zeros((r, s), _F32), x[:, :n - s]], axis=-1)
        s *= 2
    return x


def _cumsum_lanes(x):
    return _cumsum_flat(x)


def _mm_t(a, b):
    return jax.lax.dot_general(a, b, (((1,), (1,)), ((), ())),
                               preferred_element_type=_F32)


def _red2(op, x):
    return op(op(x, axis=2, keepdims=True), axis=1, keepdims=True)


def _fps_kernel(xr_ref, nx_ref, *, m):
    b = xr_ref.shape[0]
    ncol = xr_ref.shape[2]
    x0 = xr_ref[:, 0:8, :]
    x1 = xr_ref[:, 8:16, :]
    x2 = xr_ref[:, 16:24, :]
    lin = (jax.lax.broadcasted_iota(jnp.int32, (1, 8, ncol), 1) * ncol
           + jax.lax.broadcasted_iota(jnp.int32, (1, 8, ncol), 2))
    lane_m = jax.lax.broadcasted_iota(jnp.int32, (1, 1, m), 2)

    def body(i, carry):
        dists, far, nx = carry
        oh = (lin == far).astype(_F32)
        c0 = _red2(jnp.sum, x0 * oh)
        c1 = _red2(jnp.sum, x1 * oh)
        c2 = _red2(jnp.sum, x2 * oh)
        ccol = jnp.concatenate([c0, c1, c2], axis=1)
        nx = nx + ccol * (lane_m == i).astype(_F32)
        d = (x0 - c0) ** 2 + (x1 - c1) ** 2 + (x2 - c2) ** 2
        dists = jnp.minimum(dists, d)
        gmax = _red2(jnp.max, dists)
        far = _red2(jnp.min, jnp.where(dists == gmax, lin, jnp.int32(2 ** 30)))
        return dists, far, nx

    d0 = jnp.full((b, 8, ncol), 1e10, _F32)
    nx0 = jnp.zeros((b, 3, m), _F32)
    f0 = jnp.zeros((b, 1, 1), jnp.int32)
    _, _, nx = jax.lax.fori_loop(0, m, body, (d0, f0, nx0))
    nx_ref[...] = nx


def _fps_call(xyz, m):
    b, _, n = xyz.shape
    xr = jnp.reshape(xyz, (b, 24, n // 8))
    return pl.pallas_call(
        functools.partial(_fps_kernel, m=m),
        grid=(1,),
        in_specs=[pl.BlockSpec((b, 24, n // 8), lambda i: (0, 0, 0))],
        out_specs=pl.BlockSpec((b, 3, m), lambda i: (0, 0, 0)),
        out_shape=jax.ShapeDtypeStruct((b, 3, m), _F32),
    )(xr)


def _sa_kernel(xyz_ref, nx_ref, nxT_ref, feat_ref, *wref, r2, k, cin, ct,
               has_feat):
    ws = wref[:-2]
    out_ref, scr = wref[-2], wref[-1]
    xyz = xyz_ref[0]
    n = xyz.shape[1]
    nxT = nxT_ref[0]
    px0 = xyz[0:1, :]
    px1 = xyz[1:2, :]
    px2 = xyz[2:3, :]
    cx0 = nxT[:, 0:1]
    cx1 = nxT[:, 1:2]
    cx2 = nxT[:, 2:3]
    d2 = (cx0 - px0) ** 2 + (cx1 - px1) ** 2 + (cx2 - px2) ** 2
    mask = (d2 < r2).astype(_F32)
    cnt = _cumsum_lanes(mask)
    sel = mask * cnt
    tot = cnt[:, n - 1:n]
    tot_row = jax.lax.transpose(tot, (1, 0))
    nx = nx_ref[0]
    if has_feat:
        feat = feat_ref[0]

    tile_max = jnp.max(tot)

    oh0 = (sel == 1.0).astype(_F32)
    g0 = _mm_t(xyz, oh0)
    gfill = g0 - nx
    scr[0:3, 0:ct] = gfill
    if has_feat:
        f0 = _mm_t(feat, oh0)
        scr[3:, 0:ct] = f0
    else:
        f0 = None

    def slot(s, _):
        sf = s.astype(_F32)

        @pl.when(sf < tile_max)
        def _live():
            oh = (sel == sf + 1.0).astype(_F32)
            gt = _mm_t(xyz, oh)
            valid = tot_row > sf
            scr[0:3, pl.ds(s * ct, ct)] = jnp.where(valid, gt - nx, gfill)
            if has_feat:
                ft = _mm_t(feat, oh)
                scr[3:, pl.ds(s * ct, ct)] = jnp.where(valid, ft, f0)

        @pl.when(sf >= tile_max)
        def _fill():
            scr[0:3, pl.ds(s * ct, ct)] = gfill
            if has_feat:
                scr[3:, pl.ds(s * ct, ct)] = f0

        return 0

    jax.lax.fori_loop(1, k, slot, 0)

    x = scr[...]
    for wr, br in zip(ws[0::2], ws[1::2]):
        x = jnp.maximum(_mm(wr[...], x) + br[...], 0.0)
    cout = x.shape[0]
    out_ref[0] = jnp.max(x.reshape(cout, k, ct), axis=1)


def _sa_call(xyz, nx, nxT, feat, weights, r2, k, ct):
    b, _, n = xyz.shape
    m = nx.shape[2]
    has_feat = feat is not None
    cf = feat.shape[1] if has_feat else 0
    cin = 3 + cf
    cout = weights[-2].shape[0]
    nt = m // ct
    wspecs = [pl.BlockSpec(w.shape, lambda bb, t: (0, 0)) for w in weights]
    fspec = ([pl.BlockSpec((1, cf, n), lambda bb, t: (bb, 0, 0))]
             if has_feat else
             [pl.BlockSpec((1, 1, 1), lambda bb, t: (0, 0, 0))])
    farg = feat if has_feat else jnp.zeros((1, 1, 1), _F32)
    return pl.pallas_call(
        functools.partial(_sa_kernel, r2=r2, k=k, cin=cin, ct=ct,
                          has_feat=has_feat),
        grid=(b, nt),
        in_specs=[
            pl.BlockSpec((1, 3, n), lambda bb, t: (bb, 0, 0)),
            pl.BlockSpec((1, 3, ct), lambda bb, t: (bb, 0, t)),
            pl.BlockSpec((1, ct, 3), lambda bb, t: (bb, t, 0)),
        ] + fspec + wspecs,
        out_specs=pl.BlockSpec((1, cout, ct), lambda bb, t: (bb, 0, t)),
        out_shape=jax.ShapeDtypeStruct((b, cout, m), _F32),
        scratch_shapes=[pltpu.VMEM((cin, k * ct), _F32)],
        compiler_params=pltpu.CompilerParams(
            dimension_semantics=("parallel", "parallel")),
    )(xyz, nx, nxT, farg, *weights)


def _nn3_weights(d2):
    n, m = d2.shape
    wd = jnp.zeros((n, m), _F32)
    wsum = jnp.zeros((n, 1), _F32)
    lane = jax.lax.broadcasted_iota(jnp.int32, (n, m), 1)
    for _ in range(3):
        v = jnp.min(d2, axis=-1, keepdims=True)
        am = jnp.argmin(d2, axis=-1, keepdims=True)
        oh = (lane == am).astype(_F32)
        w = 1.0 / (v + 1e-8)
        wd = wd + w * oh
        wsum = wsum + w
        d2 = jnp.where(oh > 0.0, jnp.float32(np.inf), d2)
    return wd / wsum


def _d2_block(pcols, prows):
    return ((pcols[:, 0:1] - prows[0:1, :]) ** 2
            + (pcols[:, 1:2] - prows[1:2, :]) ** 2
            + (pcols[:, 2:3] - prows[2:3, :]) ** 2)


def _relu_mlp(x, ws):
    for wr, br in zip(ws[0::2], ws[1::2]):
        x = jnp.maximum(_mm(wr[...], x) + br[...], 0.0)
    return x


def _localfp1_kernel(nx2_ref, nx2T_ref, f2_ref, *refs, nl):
    lws = refs[:nl]
    pws = refs[nl:-1]
    out_ref = refs[-1]
    nx2 = nx2_ref[0]
    f2 = f2_ref[0]
    feat = jnp.concatenate([nx2, f2], axis=0)
    l3 = _relu_mlp(feat, lws)
    d2 = _d2_block(nx2T_ref[0], nx2)
    wd = _nn3_weights(d2)
    interp = _mm_t(l3, wd)
    new = jnp.concatenate([interp, f2], axis=0)
    out_ref[0] = _relu_mlp(new, pws)


def _localfp1_call(nx2, nx2T, f2, lweights, pweights):
    b = nx2.shape[0]
    m = nx2.shape[2]
    cout = pweights[-2].shape[0]
    ws = list(lweights) + list(pweights)
    wspecs = [pl.BlockSpec(w.shape, lambda bb: (0, 0)) for w in ws]
    return pl.pallas_call(
        functools.partial(_localfp1_kernel, nl=len(lweights)),
        grid=(b,),
        in_specs=[
            pl.BlockSpec((1, 3, m), lambda bb: (bb, 0, 0)),
            pl.BlockSpec((1, m, 3), lambda bb: (bb, 0, 0)),
            pl.BlockSpec((1,) + f2.shape[1:], lambda bb: (bb, 0, 0)),
        ] + wspecs,
        out_specs=pl.BlockSpec((1, cout, m), lambda bb: (bb, 0, 0)),
        out_shape=jax.ShapeDtypeStruct((b, cout, m), _F32),
        compiler_params=pltpu.CompilerParams(
            dimension_semantics=("parallel",)),
    )(nx2, nx2T, f2, *ws)


def _fp2_kernel(nx1_ref, nx1T_ref, nx2_ref, f1_ref, g1_ref, *refs):
    ws = refs[:-1]
    out_ref = refs[-1]
    d2 = _d2_block(nx1T_ref[0], nx2_ref[0])
    wd = _nn3_weights(d2)
    interp = _mm_t(g1_ref[0], wd)
    new = jnp.concatenate([interp, f1_ref[0]], axis=0)
    out_ref[0] = _relu_mlp(new, ws)


def _fp2_call(nx1, nx1T, nx2, f1, g1, weights):
    b, _, n = nx1.shape
    m = nx2.shape[2]
    cout = weights[-2].shape[0]
    wspecs = [pl.BlockSpec(w.shape, lambda bb: (0, 0)) for w in weights]
    return pl.pallas_call(
        _fp2_kernel,
        grid=(b,),
        in_specs=[
            pl.BlockSpec((1, 3, n), lambda bb: (bb, 0, 0)),
            pl.BlockSpec((1, n, 3), lambda bb: (bb, 0, 0)),
            pl.BlockSpec((1, 3, m), lambda bb: (bb, 0, 0)),
            pl.BlockSpec((1,) + f1.shape[1:], lambda bb: (bb, 0, 0)),
            pl.BlockSpec((1,) + g1.shape[1:], lambda bb: (bb, 0, 0)),
        ] + wspecs,
        out_specs=pl.BlockSpec((1, cout, n), lambda bb: (bb, 0, 0)),
        out_shape=jax.ShapeDtypeStruct((b, cout, n), _F32),
        compiler_params=pltpu.CompilerParams(
            dimension_semantics=("parallel",)),
    )(nx1, nx1T, nx2, f1, g1, *weights)


def _fp3seg_kernel(xyz_ref, xyzT_ref, nx1_ref, g2_ref, *refs, nf):
    fws = refs[:nf]
    sws = refs[nf:-3]
    lw_ref, lb_ref, out_ref = refs[-3], refs[-2], refs[-1]
    xyz = xyz_ref[0]
    d2 = _d2_block(xyzT_ref[0], nx1_ref[0])
    wd = _nn3_weights(d2)
    interp = _mm_t(g2_ref[0], wd)
    x = jnp.concatenate([interp, xyz], axis=0)
    x = _relu_mlp(x, fws)
    x = _relu_mlp(x, sws)
    out_ref[0] = _mm(lw_ref[...], x) + lb_ref[...]


def _fp3seg_call(xyz, xyzT, nx1, g2, fweights, sweights, lw, lb, pt):
    b, _, n = xyz.shape
    m = nx1.shape[2]
    nt = n // pt
    ws = list(fweights) + list(sweights) + [lw, lb]
    nseg = lw.shape[0]
    wspecs = [pl.BlockSpec(w.shape, lambda bb, t: (0, 0)) for w in ws]
    return pl.pallas_call(
        functools.partial(_fp3seg_kernel, nf=len(fweights)),
        grid=(b, nt),
        in_specs=[
            pl.BlockSpec((1, 3, pt), lambda bb, t: (bb, 0, t)),
            pl.BlockSpec((1, pt, 3), lambda bb, t: (bb, t, 0)),
            pl.BlockSpec((1, 3, m), lambda bb, t: (bb, 0, 0)),
            pl.BlockSpec((1,) + g2.shape[1:], lambda bb, t: (bb, 0, 0)),
        ] + wspecs,
        out_specs=pl.BlockSpec((1, nseg, pt), lambda bb, t: (bb, 0, t)),
        out_shape=jax.ShapeDtypeStruct((b, nseg, n), _F32),
        compiler_params=pltpu.CompilerParams(
            dimension_semantics=("parallel", "parallel")),
    )(xyz, xyzT, nx1, g2, *ws)


def _fold(layers):
    s = 1.0 / np.sqrt(1.0 + _EPS)
    out = []
    for w, g, bb in layers:
        out.append(w * (g * s)[:, None])
        out.append(bb[:, None])
    return out


def kernel(points, params):
    xyz0 = points[:, 0:3, :]
    xyz0T = jnp.transpose(xyz0, (0, 2, 1))

    sa1w = _fold(params['sa1'])
    sa2w = _fold(params['sa2'])
    locw = _fold(params['local'])
    fp1w = _fold(params['fp1'])
    fp2w = _fold(params['fp2'])
    fp3w = _fold(params['fp3'])
    segw = _fold(params['seg'])
    lw = params['logit_w']
    lb = params['logit_b'][:, None]

    nx1 = _fps_call(xyz0, _M1)
    nx1T = jnp.transpose(nx1, (0, 2, 1))
    f1 = _sa_call(xyz0, nx1, nx1T, None, sa1w, _R1SQ, _K, _CTILE)
    nx2 = _fps_call(nx1, _M2)
    nx2T = jnp.transpose(nx2, (0, 2, 1))
    f2 = _sa_call(nx1, nx2, nx2T, f1, sa2w, _R2SQ, _K, _M2)
    g1 = _localfp1_call(nx2, nx2T, f2, locw, fp1w)
    g2 = _fp2_call(nx1, nx1T, nx2, f1, g1, fp2w)
    return _fp3seg_call(xyz0, xyz0T, nx1, g2, fp3w, segw, lw, lb, _PTILE)


# speedup vs baseline: 2.5235x; 1.0420x over previous
import functools

import jax
import jax.numpy as jnp
import numpy as np
from jax.experimental import pallas as pl
from jax.experimental.pallas import tpu as pltpu

_EPS = 1e-5
_R1SQ = 0.2 * 0.2
_R2SQ = 0.4 * 0.4
_K = 64
_M1 = 512
_M2 = 128
_CTILE = 128
_PTILE = 4096
_F32 = jnp.float32


def _mm(w, x):
    return jax.lax.dot_general(w, x, (((1,), (0,)), ((), ())),
                               preferred_element_type=_F32)


def _cumsum_flat(x):
    r, n = x.shape
    s = 1
    while s < n:
        x = x + jnp.concatenate(
            [jnp.zeros((r, s), _F32), x[:, :n - s]], axis=-1)
        s *= 2
    return x


def _cumsum_lanes(x):
    return _cumsum_flat(x)


def _mm_t(a, b):
    return jax.lax.dot_general(a, b, (((1,), (1,)), ((), ())),
                               preferred_element_type=_F32)


def _red2(op, x):
    return op(op(x, axis=2, keepdims=True), axis=1, keepdims=True)


def _fps_kernel(xr_ref, nx_ref, *, m):
    b = xr_ref.shape[0]
    ncol = xr_ref.shape[2]
    x0 = xr_ref[:, 0:8, :]
    x1 = xr_ref[:, 8:16, :]
    x2 = xr_ref[:, 16:24, :]
    lin = (jax.lax.broadcasted_iota(jnp.int32, (1, 8, ncol), 1) * ncol
           + jax.lax.broadcasted_iota(jnp.int32, (1, 8, ncol), 2))
    lane_m = jax.lax.broadcasted_iota(jnp.int32, (1, 1, m), 2)

    def body(i, carry):
        dists, far, nx = carry
        oh = (lin == far).astype(_F32)
        c0 = _red2(jnp.sum, x0 * oh)
        c1 = _red2(jnp.sum, x1 * oh)
        c2 = _red2(jnp.sum, x2 * oh)
        ccol = jnp.concatenate([c0, c1, c2], axis=1)
        nx = nx + ccol * (lane_m == i).astype(_F32)
        d = (x0 - c0) ** 2 + (x1 - c1) ** 2 + (x2 - c2) ** 2
        dists = jnp.minimum(dists, d)
        gmax = _red2(jnp.max, dists)
        far = _red2(jnp.min, jnp.where(dists == gmax, lin, jnp.int32(2 ** 30)))
        return dists, far, nx

    d0 = jnp.full((b, 8, ncol), 1e10, _F32)
    nx0 = jnp.zeros((b, 3, m), _F32)
    f0 = jnp.zeros((b, 1, 1), jnp.int32)
    _, _, nx = jax.lax.fori_loop(0, m, body, (d0, f0, nx0))
    nx_ref[...] = nx


def _fps_call(xyz, m):
    b, _, n = xyz.shape
    xr = jnp.reshape(xyz, (b, 24, n // 8))
    return pl.pallas_call(
        functools.partial(_fps_kernel, m=m),
        grid=(1,),
        in_specs=[pl.BlockSpec((b, 24, n // 8), lambda i: (0, 0, 0))],
        out_specs=pl.BlockSpec((b, 3, m), lambda i: (0, 0, 0)),
        out_shape=jax.ShapeDtypeStruct((b, 3, m), _F32),
    )(xr)


def _sa_kernel(xyz_ref, nx_ref, nxT_ref, feat_ref, *wref, r2, k, cin, ct,
               has_feat):
    ws = wref[:-2]
    out_ref, scr = wref[-2], wref[-1]
    xyz = xyz_ref[0]
    n = xyz.shape[1]
    nxT = nxT_ref[0]
    px0 = xyz[0:1, :]
    px1 = xyz[1:2, :]
    px2 = xyz[2:3, :]
    cx0 = nxT[:, 0:1]
    cx1 = nxT[:, 1:2]
    cx2 = nxT[:, 2:3]
    d2 = (cx0 - px0) ** 2 + (cx1 - px1) ** 2 + (cx2 - px2) ** 2
    mask = (d2 < r2).astype(_F32)
    cnt = _cumsum_lanes(mask)
    sel = mask * cnt
    tot = cnt[:, n - 1:n]
    tot_row = jax.lax.transpose(tot, (1, 0))
    nx = nx_ref[0]
    if has_feat:
        feat = feat_ref[0]

    tile_max = jnp.max(tot)

    oh0 = (sel == 1.0).astype(_F32)
    g0 = _mm_t(xyz, oh0)
    gfill = g0 - nx
    scr[0:3, 0:ct] = gfill
    if has_feat:
        f0 = _mm_t(feat, oh0)
        scr[3:, 0:ct] = f0
    else:
        f0 = None

    def slot(s, _):
        sf = s.astype(_F32)

        @pl.when(sf < tile_max)
        def _live():
            oh = (sel == sf + 1.0).astype(_F32)
            gt = _mm_t(xyz, oh)
            valid = tot_row > sf
            scr[0:3, pl.ds(s * ct, ct)] = jnp.where(valid, gt - nx, gfill)
            if has_feat:
                ft = _mm_t(feat, oh)
                scr[3:, pl.ds(s * ct, ct)] = jnp.where(valid, ft, f0)

        @pl.when(sf >= tile_max)
        def _fill():
            scr[0:3, pl.ds(s * ct, ct)] = gfill
            if has_feat:
                scr[3:, pl.ds(s * ct, ct)] = f0

        return 0

    jax.lax.fori_loop(1, k, slot, 0)

    x = scr[...]
    for wr, br in zip(ws[0::2], ws[1::2]):
        x = jnp.maximum(_mm(wr[...], x) + br[...], 0.0)
    cout = x.shape[0]
    out_ref[0] = jnp.max(x.reshape(cout, k, ct), axis=1)


def _sa_call(xyz, nx, nxT, feat, weights, r2, k, ct):
    b, _, n = xyz.shape
    m = nx.shape[2]
    has_feat = feat is not None
    cf = feat.shape[1] if has_feat else 0
    cin = 3 + cf
    cout = weights[-2].shape[0]
    nt = m // ct
    wspecs = [pl.BlockSpec(w.shape, lambda bb, t: (0, 0)) for w in weights]
    fspec = ([pl.BlockSpec((1, cf, n), lambda bb, t: (bb, 0, 0))]
             if has_feat else
             [pl.BlockSpec((1, 1, 1), lambda bb, t: (0, 0, 0))])
    farg = feat if has_feat else jnp.zeros((1, 1, 1), _F32)
    return pl.pallas_call(
        functools.partial(_sa_kernel, r2=r2, k=k, cin=cin, ct=ct,
                          has_feat=has_feat),
        grid=(b, nt),
        in_specs=[
            pl.BlockSpec((1, 3, n), lambda bb, t: (bb, 0, 0)),
            pl.BlockSpec((1, 3, ct), lambda bb, t: (bb, 0, t)),
            pl.BlockSpec((1, ct, 3), lambda bb, t: (bb, t, 0)),
        ] + fspec + wspecs,
        out_specs=pl.BlockSpec((1, cout, ct), lambda bb, t: (bb, 0, t)),
        out_shape=jax.ShapeDtypeStruct((b, cout, m), _F32),
        scratch_shapes=[pltpu.VMEM((cin, k * ct), _F32)],
        compiler_params=pltpu.CompilerParams(
            dimension_semantics=("parallel", "parallel")),
    )(xyz, nx, nxT, farg, *weights)


def _nn3_weights(d2):
    n, m = d2.shape
    wd = jnp.zeros((n, m), _F32)
    wsum = jnp.zeros((n, 1), _F32)
    lane = jax.lax.broadcasted_iota(jnp.int32, (n, m), 1)
    for _ in range(3):
        v = jnp.min(d2, axis=-1, keepdims=True)
        am = jnp.argmin(d2, axis=-1, keepdims=True)
        oh = (lane == am).astype(_F32)
        w = 1.0 / (v + 1e-8)
        wd = wd + w * oh
        wsum = wsum + w
        d2 = jnp.where(oh > 0.0, jnp.float32(np.inf), d2)
    return wd / wsum


def _d2_block(pcols, prows):
    return ((pcols[:, 0:1] - prows[0:1, :]) ** 2
            + (pcols[:, 1:2] - prows[1:2, :]) ** 2
            + (pcols[:, 2:3] - prows[2:3, :]) ** 2)


def _relu_mlp(x, ws):
    for wr, br in zip(ws[0::2], ws[1::2]):
        x = jnp.maximum(_mm(wr[...], x) + br[...], 0.0)
    return x


def _localfp1_kernel(nx2_ref, nx2T_ref, f2_ref, *refs, nl):
    lws = refs[:nl]
    pws = refs[nl:-1]
    out_ref = refs[-1]
    nx2 = nx2_ref[0]
    f2 = f2_ref[0]
    feat = jnp.concatenate([nx2, f2], axis=0)
    l3 = _relu_mlp(feat, lws)
    d2 = _d2_block(nx2T_ref[0], nx2)
    wd = _nn3_weights(d2)
    interp = _mm_t(l3, wd)
    new = jnp.concatenate([interp, f2], axis=0)
    out_ref[0] = _relu_mlp(new, pws)


def _localfp1_call(nx2, nx2T, f2, lweights, pweights):
    b = nx2.shape[0]
    m = nx2.shape[2]
    cout = pweights[-2].shape[0]
    ws = list(lweights) + list(pweights)
    wspecs = [pl.BlockSpec(w.shape, lambda bb: (0, 0)) for w in ws]
    return pl.pallas_call(
        functools.partial(_localfp1_kernel, nl=len(lweights)),
        grid=(b,),
        in_specs=[
            pl.BlockSpec((1, 3, m), lambda bb: (bb, 0, 0)),
            pl.BlockSpec((1, m, 3), lambda bb: (bb, 0, 0)),
            pl.BlockSpec((1,) + f2.shape[1:], lambda bb: (bb, 0, 0)),
        ] + wspecs,
        out_specs=pl.BlockSpec((1, cout, m), lambda bb: (bb, 0, 0)),
        out_shape=jax.ShapeDtypeStruct((b, cout, m), _F32),
        compiler_params=pltpu.CompilerParams(
            dimension_semantics=("parallel",)),
    )(nx2, nx2T, f2, *ws)


def _fp2_kernel(nx1_ref, nx1T_ref, nx2_ref, f1_ref, g1_ref, *refs):
    ws = refs[:-1]
    out_ref = refs[-1]
    d2 = _d2_block(nx1T_ref[0], nx2_ref[0])
    wd = _nn3_weights(d2)
    interp = _mm_t(g1_ref[0], wd)
    new = jnp.concatenate([interp, f1_ref[0]], axis=0)
    out_ref[0] = _relu_mlp(new, ws)


def _fp2_call(nx1, nx1T, nx2, f1, g1, weights):
    b, _, n = nx1.shape
    m = nx2.shape[2]
    cout = weights[-2].shape[0]
    wspecs = [pl.BlockSpec(w.shape, lambda bb: (0, 0)) for w in weights]
    return pl.pallas_call(
        _fp2_kernel,
        grid=(b,),
        in_specs=[
            pl.BlockSpec((1, 3, n), lambda bb: (bb, 0, 0)),
            pl.BlockSpec((1, n, 3), lambda bb: (bb, 0, 0)),
            pl.BlockSpec((1, 3, m), lambda bb: (bb, 0, 0)),
            pl.BlockSpec((1,) + f1.shape[1:], lambda bb: (bb, 0, 0)),
            pl.BlockSpec((1,) + g1.shape[1:], lambda bb: (bb, 0, 0)),
        ] + wspecs,
        out_specs=pl.BlockSpec((1, cout, n), lambda bb: (bb, 0, 0)),
        out_shape=jax.ShapeDtypeStruct((b, cout, n), _F32),
        compiler_params=pltpu.CompilerParams(
            dimension_semantics=("parallel",)),
    )(nx1, nx1T, nx2, f1, g1, *weights)


def _fp3seg_kernel(xyz_ref, xyzT_ref, nx1_ref, g2_ref, *refs, nf):
    fws = refs[:nf]
    sws = refs[nf:-3]
    lw_ref, lb_ref, out_ref = refs[-3], refs[-2], refs[-1]
    xyz = xyz_ref[0]
    d2 = _d2_block(xyzT_ref[0], nx1_ref[0])
    wd = _nn3_weights(d2)
    interp = _mm_t(g2_ref[0], wd)
    x = jnp.concatenate([interp, xyz], axis=0)
    x = _relu_mlp(x, fws)
    x = _relu_mlp(x, sws)
    out_ref[0] = _mm(lw_ref[...], x) + lb_ref[...]


def _fp3seg_call(xyz, xyzT, nx1, g2, fweights, sweights, lw, lb, pt):
    b, _, n = xyz.shape
    m = nx1.shape[2]
    nt = n // pt
    ws = list(fweights) + list(sweights) + [lw, lb]
    nseg = lw.shape[0]
    wspecs = [pl.BlockSpec(w.shape, lambda bb, t: (0, 0)) for w in ws]
    return pl.pallas_call(
        functools.partial(_fp3seg_kernel, nf=len(fweights)),
        grid=(b, nt),
        in_specs=[
            pl.BlockSpec((1, 3, pt), lambda bb, t: (bb, 0, t)),
            pl.BlockSpec((1, pt, 3), lambda bb, t: (bb, t, 0)),
            pl.BlockSpec((1, 3, m), lambda bb, t: (bb, 0, 0)),
            pl.BlockSpec((1,) + g2.shape[1:], lambda bb, t: (bb, 0, 0)),
        ] + wspecs,
        out_specs=pl.BlockSpec((1, nseg, pt), lambda bb, t: (bb, 0, t)),
        out_shape=jax.ShapeDtypeStruct((b, nseg, n), _F32),
        compiler_params=pltpu.CompilerParams(
            dimension_semantics=("parallel", "parallel")),
    )(xyz, xyzT, nx1, g2, *ws)


def _fold(layers):
    s = 1.0 / np.sqrt(1.0 + _EPS)
    out = []
    for w, g, bb in layers:
        out.append(w * (g * s)[:, None])
        out.append(bb[:, None])
    return out


def kernel(points, params):
    xyz0 = points[:, 0:3, :]
    xyz0T = jnp.transpose(xyz0, (0, 2, 1))

    sa1w = _fold(params['sa1'])
    sa2w = _fold(params['sa2'])
    locw = _fold(params['local'])
    fp1w = _fold(params['fp1'])
    fp2w = _fold(params['fp2'])
    fp3w = _fold(params['fp3'])
    segw = _fold(params['seg'])
    lw = params['logit_w']
    lb = params['logit_b'][:, None]

    nx1 = _fps_call(xyz0, _M1)
    nx1T = jnp.transpose(nx1, (0, 2, 1))
    f1 = _sa_call(xyz0, nx1, nx1T, None, sa1w, _R1SQ, _K, _CTILE)
    nx2 = _fps_call(nx1, _M2)
    nx2T = jnp.transpose(nx2, (0, 2, 1))
    f2 = _sa_call(nx1, nx2, nx2T, f1, sa2w, _R2SQ, _K, _M2)
    g1 = _localfp1_call(nx2, nx2T, f2, locw, fp1w)
    g2 = _fp2_call(nx1, nx1T, nx2, f1, g1, fp2w)
    return _fp3seg_call(xyz0, xyz0T, nx1, g2, fp3w, segw, lw, lb, _PTILE)
